# baseline (device time: 94956 ns/iter reference)
import jax
import jax.numpy as jnp
from jax import lax
from jax.experimental import pallas as pl
from jax.experimental.pallas import tpu as pltpu

N_DEV = 16
N_TOK = 1024
D_IN = 512
D_OUT = 1024
E_LOCAL = 4
ROWS = N_TOK // N_DEV


def kernel(x, router_W, route_idx, expert_W, shared_W):
    probs = jax.nn.softmax(x @ router_W, axis=-1)
    coef = jnp.take_along_axis(probs, route_idx, axis=-1)
    my_i = lax.axis_index("i")
    e_ids = my_i * E_LOCAL + jnp.arange(E_LOCAL, dtype=route_idx.dtype)
    scale = jnp.where(route_idx == e_ids[None, :], coef, 0.0)

    def body(x_ref, ew_ref, sw_ref, scale_ref, out_ref,
             partial_ref, recv_ref, send_sems, recv_sems):
        d = lax.axis_index("i")
        left = lax.rem(d + N_DEV - 1, N_DEV)
        right = lax.rem(d + 1, N_DEV)

        barrier = pltpu.get_barrier_semaphore()
        for nbr in (left, right):
            pl.semaphore_signal(barrier, inc=1, device_id=(nbr,),
                                device_id_type=pl.DeviceIdType.MESH)
        pl.semaphore_wait(barrier, 2)

        acc = (x_ref[:, :] * scale_ref[:, 0:1]) @ ew_ref[0]
        for e in range(1, E_LOCAL):
            acc += (x_ref[:, :] * scale_ref[:, e:e + 1]) @ ew_ref[e]
        partial_ref[:, :] = acc

        for s in range(N_DEV - 1):
            send_chunk = lax.rem(d + 2 * N_DEV - 1 - s, N_DEV)
            recv_chunk = lax.rem(d + 2 * N_DEV - 2 - s, N_DEV)
            rdma = pltpu.make_async_remote_copy(
                src_ref=partial_ref.at[pl.ds(send_chunk * ROWS, ROWS), :],
                dst_ref=recv_ref.at[s],
                send_sem=send_sems.at[s],
                recv_sem=recv_sems.at[s],
                device_id=(right,),
                device_id_type=pl.DeviceIdType.MESH,
            )
            rdma.start()
            rdma.wait()
            partial_ref[pl.ds(recv_chunk * ROWS, ROWS), :] += recv_ref[s]

        shared = x_ref[pl.ds(d * ROWS, ROWS), :] @ sw_ref[:, :]
        out_ref[:, :] = partial_ref[pl.ds(d * ROWS, ROWS), :] + shared

    return pl.pallas_call(
        body,
        out_shape=jax.ShapeDtypeStruct((ROWS, D_OUT), jnp.float32),
        in_specs=[pl.BlockSpec(memory_space=pltpu.VMEM)] * 4,
        out_specs=pl.BlockSpec(memory_space=pltpu.VMEM),
        scratch_shapes=[
            pltpu.VMEM((N_TOK, D_OUT), jnp.float32),
            pltpu.VMEM((N_DEV - 1, ROWS, D_OUT), jnp.float32),
            pltpu.SemaphoreType.DMA((N_DEV - 1,)),
            pltpu.SemaphoreType.DMA((N_DEV - 1,)),
        ],
        compiler_params=pltpu.CompilerParams(collective_id=0),
    )(x, expert_W, shared_W, scale)


# device time: 59131 ns/iter; 1.6059x vs baseline; 1.6059x over previous
import jax
import jax.numpy as jnp
from jax import lax
from jax.experimental import pallas as pl
from jax.experimental.pallas import tpu as pltpu

N_DEV = 16
PLANE = 4
N_Z = 4
N_TOK = 1024
D_IN = 512
D_OUT = 1024
E_LOCAL = 4
ROWS = N_TOK // N_DEV


def kernel(x, router_W, route_idx, expert_W, shared_W):
    probs = jax.nn.softmax(x @ router_W, axis=-1)
    coef = jnp.take_along_axis(probs, route_idx, axis=-1)
    my_i = lax.axis_index("i")
    e_ids = my_i * E_LOCAL + jnp.arange(E_LOCAL, dtype=route_idx.dtype)
    scale = jnp.where(route_idx == e_ids[None, :], coef, 0.0)

    def body(x_ref, ew_ref, sw_ref, scale_ref, out_ref,
             partial_ref, rp1_ref, rp2_ref,
             p1_send_sems, p1_recv_sems, p2_send_sems, p2_recv_sems):
        d = lax.axis_index("i")
        my_z = lax.div(d, PLANE)
        my_w = lax.rem(d, PLANE)

        barrier = pltpu.get_barrier_semaphore()
        for o in range(1, PLANE):
            wp = lax.rem(my_w + o, PLANE)
            pl.semaphore_signal(barrier, inc=1, device_id=(my_z * PLANE + wp,),
                                device_id_type=pl.DeviceIdType.MESH)
        for o in range(1, N_Z):
            zq = lax.rem(my_z + o, N_Z)
            pl.semaphore_signal(barrier, inc=1, device_id=(zq * PLANE + my_w,),
                                device_id_type=pl.DeviceIdType.MESH)
        pl.semaphore_wait(barrier, 6)

        acc = (x_ref[:, :] * scale_ref[:, 0:1]) @ ew_ref[0]
        for e in range(1, E_LOCAL):
            acc += (x_ref[:, :] * scale_ref[:, e:e + 1]) @ ew_ref[e]
        partial_ref[:, :] = acc

        sends = []

        for o in range(1, PLANE):
            wp = lax.rem(my_w + o, PLANE)
            dest = my_z * PLANE + wp
            for zk in range(N_Z):
                chunk = 4 * zk + wp
                rdma = pltpu.make_async_remote_copy(
                    src_ref=partial_ref.at[pl.ds(chunk * ROWS, ROWS), :],
                    dst_ref=rp1_ref.at[my_w, zk],
                    send_sem=p1_send_sems.at[wp, zk],
                    recv_sem=p1_recv_sems.at[my_w, zk],
                    device_id=(dest,),
                    device_id_type=pl.DeviceIdType.MESH,
                )
                rdma.start()
                sends.append(rdma)

        def accum_zgroup(zk):
            chunk = 4 * zk + my_w
            for oo in range(1, PLANE):
                wq = lax.rem(my_w + oo, PLANE)
                recv = pltpu.make_async_remote_copy(
                    src_ref=partial_ref.at[pl.ds(0, ROWS), :],
                    dst_ref=rp1_ref.at[wq, zk],
                    send_sem=p1_send_sems.at[wq, zk],
                    recv_sem=p1_recv_sems.at[wq, zk],
                    device_id=(d,),
                    device_id_type=pl.DeviceIdType.MESH,
                )
                recv.wait_recv()
                partial_ref[pl.ds(chunk * ROWS, ROWS), :] += rp1_ref[wq, zk]

        for o in range(1, N_Z):
            zq = lax.rem(my_z + o, N_Z)
            accum_zgroup(zq)
            dest = zq * PLANE + my_w
            rdma = pltpu.make_async_remote_copy(
                src_ref=partial_ref.at[pl.ds(dest * ROWS, ROWS), :],
                dst_ref=rp2_ref.at[my_z],
                send_sem=p2_send_sems.at[zq],
                recv_sem=p2_recv_sems.at[my_z],
                device_id=(dest,),
                device_id_type=pl.DeviceIdType.MESH,
            )
            rdma.start()
            sends.append(rdma)

        accum_zgroup_own = lax.rem(my_z, N_Z)
        accum_zgroup(accum_zgroup_own)

        shared = x_ref[pl.ds(d * ROWS, ROWS), :] @ sw_ref[:, :]
        result = partial_ref[pl.ds(d * ROWS, ROWS), :] + shared

        for o in range(1, N_Z):
            zr = lax.rem(my_z + o, N_Z)
            recv = pltpu.make_async_remote_copy(
                src_ref=partial_ref.at[pl.ds(0, ROWS), :],
                dst_ref=rp2_ref.at[zr],
                send_sem=p2_send_sems.at[zr],
                recv_sem=p2_recv_sems.at[zr],
                device_id=(d,),
                device_id_type=pl.DeviceIdType.MESH,
            )
            recv.wait_recv()
            result += rp2_ref[zr]
        out_ref[:, :] = result

        for rdma in sends:
            rdma.wait_send()

    return pl.pallas_call(
        body,
        out_shape=jax.ShapeDtypeStruct((ROWS, D_OUT), jnp.float32),
        in_specs=[pl.BlockSpec(memory_space=pltpu.VMEM)] * 4,
        out_specs=pl.BlockSpec(memory_space=pltpu.VMEM),
        scratch_shapes=[
            pltpu.VMEM((N_TOK, D_OUT), jnp.float32),
            pltpu.VMEM((PLANE, N_Z, ROWS, D_OUT), jnp.float32),
            pltpu.VMEM((N_Z, ROWS, D_OUT), jnp.float32),
            pltpu.SemaphoreType.DMA((PLANE, N_Z)),
            pltpu.SemaphoreType.DMA((PLANE, N_Z)),
            pltpu.SemaphoreType.DMA((N_Z,)),
            pltpu.SemaphoreType.DMA((N_Z,)),
        ],
        compiler_params=pltpu.CompilerParams(collective_id=0),
    )(x, expert_W, shared_W, scale)


# device time: 49023 ns/iter; 1.9370x vs baseline; 1.2062x over previous
import jax
import jax.numpy as jnp
from jax import lax
from jax.experimental import pallas as pl
from jax.experimental.pallas import tpu as pltpu

N_DEV = 16
PLANE = 4
N_Z = 4
N_TOK = 1024
D_IN = 512
D_OUT = 1024
E_LOCAL = 4
ROWS = N_TOK // N_DEV


def kernel(x, router_W, route_idx, expert_W, shared_W):
    probs = jax.nn.softmax(x @ router_W, axis=-1)
    coef = jnp.take_along_axis(probs, route_idx, axis=-1)
    my_i = lax.axis_index("i")
    e_ids = my_i * E_LOCAL + jnp.arange(E_LOCAL, dtype=route_idx.dtype)
    scale = jnp.where(route_idx == e_ids[None, :], coef, 0.0)

    def body(x_ref, ew_ref, sw_ref, scale_ref, out_ref,
             partial_ref, rp1_ref, rp2_ref,
             p1_send_sems, p1_recv_sems, p2_send_sems, p2_recv_sems):
        d = lax.axis_index("i")
        my_z = lax.div(d, PLANE)
        my_w = lax.rem(d, PLANE)

        barrier = pltpu.get_barrier_semaphore()
        for o in range(1, PLANE):
            wp = lax.rem(my_w + o, PLANE)
            pl.semaphore_signal(barrier, inc=1, device_id=(my_z * PLANE + wp,),
                                device_id_type=pl.DeviceIdType.MESH)
        for o in range(1, N_Z):
            zq = lax.rem(my_z + o, N_Z)
            pl.semaphore_signal(barrier, inc=1, device_id=(zq * PLANE + my_w,),
                                device_id_type=pl.DeviceIdType.MESH)
        pl.semaphore_wait(barrier, 6)

        sends = []

        BLK = PLANE * ROWS
        for k in range(1, N_Z + 1):
            zk = lax.rem(my_z + k, N_Z)
            rs = zk * BLK
            acc = (x_ref[pl.ds(rs, BLK), :] * scale_ref[pl.ds(rs, BLK), 0:1]) @ ew_ref[0]
            for e in range(1, E_LOCAL):
                acc += (x_ref[pl.ds(rs, BLK), :] * scale_ref[pl.ds(rs, BLK), e:e + 1]) @ ew_ref[e]
            partial_ref[pl.ds(rs, BLK), :] = acc
            for o in range(1, PLANE):
                wp = lax.rem(my_w + o, PLANE)
                dest = my_z * PLANE + wp
                chunk = 4 * zk + wp
                rdma = pltpu.make_async_remote_copy(
                    src_ref=partial_ref.at[pl.ds(chunk * ROWS, ROWS), :],
                    dst_ref=rp1_ref.at[my_w, zk],
                    send_sem=p1_send_sems.at[wp, zk],
                    recv_sem=p1_recv_sems.at[my_w, zk],
                    device_id=(dest,),
                    device_id_type=pl.DeviceIdType.MESH,
                )
                rdma.start()
                sends.append(rdma)

        def accum_zgroup(zk):
            chunk = 4 * zk + my_w
            for oo in range(1, PLANE):
                wq = lax.rem(my_w + oo, PLANE)
                recv = pltpu.make_async_remote_copy(
                    src_ref=partial_ref.at[pl.ds(0, ROWS), :],
                    dst_ref=rp1_ref.at[wq, zk],
                    send_sem=p1_send_sems.at[wq, zk],
                    recv_sem=p1_recv_sems.at[wq, zk],
                    device_id=(d,),
                    device_id_type=pl.DeviceIdType.MESH,
                )
                recv.wait_recv()
                partial_ref[pl.ds(chunk * ROWS, ROWS), :] += rp1_ref[wq, zk]

        for o in range(1, N_Z):
            zq = lax.rem(my_z + o, N_Z)
            accum_zgroup(zq)
            dest = zq * PLANE + my_w
            rdma = pltpu.make_async_remote_copy(
                src_ref=partial_ref.at[pl.ds(dest * ROWS, ROWS), :],
                dst_ref=rp2_ref.at[my_z],
                send_sem=p2_send_sems.at[zq],
                recv_sem=p2_recv_sems.at[my_z],
                device_id=(dest,),
                device_id_type=pl.DeviceIdType.MESH,
            )
            rdma.start()
            sends.append(rdma)

        accum_zgroup_own = lax.rem(my_z, N_Z)
        accum_zgroup(accum_zgroup_own)

        shared = x_ref[pl.ds(d * ROWS, ROWS), :] @ sw_ref[:, :]
        result = partial_ref[pl.ds(d * ROWS, ROWS), :] + shared

        for o in range(1, N_Z):
            zr = lax.rem(my_z + o, N_Z)
            recv = pltpu.make_async_remote_copy(
                src_ref=partial_ref.at[pl.ds(0, ROWS), :],
                dst_ref=rp2_ref.at[zr],
                send_sem=p2_send_sems.at[zr],
                recv_sem=p2_recv_sems.at[zr],
                device_id=(d,),
                device_id_type=pl.DeviceIdType.MESH,
            )
            recv.wait_recv()
            result += rp2_ref[zr]
        out_ref[:, :] = result

        for rdma in sends:
            rdma.wait_send()

    return pl.pallas_call(
        body,
        out_shape=jax.ShapeDtypeStruct((ROWS, D_OUT), jnp.float32),
        in_specs=[pl.BlockSpec(memory_space=pltpu.VMEM)] * 4,
        out_specs=pl.BlockSpec(memory_space=pltpu.VMEM),
        scratch_shapes=[
            pltpu.VMEM((N_TOK, D_OUT), jnp.float32),
            pltpu.VMEM((PLANE, N_Z, ROWS, D_OUT), jnp.float32),
            pltpu.VMEM((N_Z, ROWS, D_OUT), jnp.float32),
            pltpu.SemaphoreType.DMA((PLANE, N_Z)),
            pltpu.SemaphoreType.DMA((PLANE, N_Z)),
            pltpu.SemaphoreType.DMA((N_Z,)),
            pltpu.SemaphoreType.DMA((N_Z,)),
        ],
        compiler_params=pltpu.CompilerParams(collective_id=0),
    )(x, expert_W, shared_W, scale)


# device time: 45083 ns/iter; 2.1062x vs baseline; 1.0874x over previous
import jax
import jax.numpy as jnp
from jax import lax
from jax.experimental import pallas as pl
from jax.experimental.pallas import tpu as pltpu

N_DEV = 16
PLANE = 4
N_Z = 4
N_TOK = 1024
D_IN = 512
D_OUT = 1024
E_LOCAL = 4
ROWS = N_TOK // N_DEV


def kernel(x, router_W, route_idx, expert_W, shared_W):
    def body(x_ref, rw_ref, ri_ref, ew_ref, sw_ref, out_ref,
             partial_ref, rp1_ref, rp2_ref, scale_ref,
             p1_send_sems, p1_recv_sems, p2_send_sems, p2_recv_sems):
        d = lax.axis_index("i")
        my_z = lax.div(d, PLANE)
        my_w = lax.rem(d, PLANE)

        scores = x_ref[:, :] @ rw_ref[:, :]
        m = jnp.max(scores, axis=-1, keepdims=True)
        p = jnp.exp(scores - m)
        probs = p / jnp.sum(p, axis=-1, keepdims=True)
        route = ri_ref[:, :]
        eids = lax.broadcasted_iota(route.dtype, scores.shape, 1)
        coef = jnp.sum(jnp.where(eids == route, probs, 0.0),
                       axis=-1, keepdims=True)
        scale_ref[:, :] = jnp.concatenate(
            [jnp.where(route == d * E_LOCAL + e, coef, 0.0)
             for e in range(E_LOCAL)], axis=-1)

        barrier = pltpu.get_barrier_semaphore()
        for o in range(1, PLANE):
            wp = lax.rem(my_w + o, PLANE)
            pl.semaphore_signal(barrier, inc=1, device_id=(my_z * PLANE + wp,),
                                device_id_type=pl.DeviceIdType.MESH)
        for o in range(1, N_Z):
            zq = lax.rem(my_z + o, N_Z)
            pl.semaphore_signal(barrier, inc=1, device_id=(zq * PLANE + my_w,),
                                device_id_type=pl.DeviceIdType.MESH)
        pl.semaphore_wait(barrier, 6)

        sends = []

        BLK = PLANE * ROWS
        for k in range(1, N_Z + 1):
            zk = lax.rem(my_z + k, N_Z)
            rs = zk * BLK
            acc = (x_ref[pl.ds(rs, BLK), :] * scale_ref[pl.ds(rs, BLK), 0:1]) @ ew_ref[0]
            for e in range(1, E_LOCAL):
                acc += (x_ref[pl.ds(rs, BLK), :] * scale_ref[pl.ds(rs, BLK), e:e + 1]) @ ew_ref[e]
            partial_ref[pl.ds(rs, BLK), :] = acc
            for o in range(1, PLANE):
                wp = lax.rem(my_w + o, PLANE)
                dest = my_z * PLANE + wp
                chunk = 4 * zk + wp
                rdma = pltpu.make_async_remote_copy(
                    src_ref=partial_ref.at[pl.ds(chunk * ROWS, ROWS), :],
                    dst_ref=rp1_ref.at[my_w, zk],
                    send_sem=p1_send_sems.at[wp, zk],
                    recv_sem=p1_recv_sems.at[my_w, zk],
                    device_id=(dest,),
                    device_id_type=pl.DeviceIdType.MESH,
                )
                rdma.start()
                sends.append(rdma)

        def accum_zgroup(zk):
            chunk = 4 * zk + my_w
            for oo in range(1, PLANE):
                wq = lax.rem(my_w + oo, PLANE)
                recv = pltpu.make_async_remote_copy(
                    src_ref=partial_ref.at[pl.ds(0, ROWS), :],
                    dst_ref=rp1_ref.at[wq, zk],
                    send_sem=p1_send_sems.at[wq, zk],
                    recv_sem=p1_recv_sems.at[wq, zk],
                    device_id=(d,),
                    device_id_type=pl.DeviceIdType.MESH,
                )
                recv.wait_recv()
                partial_ref[pl.ds(chunk * ROWS, ROWS), :] += rp1_ref[wq, zk]

        for o in range(1, N_Z):
            zq = lax.rem(my_z + o, N_Z)
            accum_zgroup(zq)
            dest = zq * PLANE + my_w
            rdma = pltpu.make_async_remote_copy(
                src_ref=partial_ref.at[pl.ds(dest * ROWS, ROWS), :],
                dst_ref=rp2_ref.at[my_z],
                send_sem=p2_send_sems.at[zq],
                recv_sem=p2_recv_sems.at[my_z],
                device_id=(dest,),
                device_id_type=pl.DeviceIdType.MESH,
            )
            rdma.start()
            sends.append(rdma)

        accum_zgroup_own = lax.rem(my_z, N_Z)
        accum_zgroup(accum_zgroup_own)

        shared = x_ref[pl.ds(d * ROWS, ROWS), :] @ sw_ref[:, :]
        result = partial_ref[pl.ds(d * ROWS, ROWS), :] + shared

        for o in range(1, N_Z):
            zr = lax.rem(my_z + o, N_Z)
            recv = pltpu.make_async_remote_copy(
                src_ref=partial_ref.at[pl.ds(0, ROWS), :],
                dst_ref=rp2_ref.at[zr],
                send_sem=p2_send_sems.at[zr],
                recv_sem=p2_recv_sems.at[zr],
                device_id=(d,),
                device_id_type=pl.DeviceIdType.MESH,
            )
            recv.wait_recv()
            result += rp2_ref[zr]
        out_ref[:, :] = result

        for rdma in sends:
            rdma.wait_send()

    return pl.pallas_call(
        body,
        out_shape=jax.ShapeDtypeStruct((ROWS, D_OUT), jnp.float32),
        in_specs=[pl.BlockSpec(memory_space=pltpu.VMEM)] * 5,
        out_specs=pl.BlockSpec(memory_space=pltpu.VMEM),
        scratch_shapes=[
            pltpu.VMEM((N_TOK, D_OUT), jnp.float32),
            pltpu.VMEM((PLANE, N_Z, ROWS, D_OUT), jnp.float32),
            pltpu.VMEM((N_Z, ROWS, D_OUT), jnp.float32),
            pltpu.VMEM((N_TOK, E_LOCAL), jnp.float32),
            pltpu.SemaphoreType.DMA((PLANE, N_Z)),
            pltpu.SemaphoreType.DMA((PLANE, N_Z)),
            pltpu.SemaphoreType.DMA((N_Z,)),
            pltpu.SemaphoreType.DMA((N_Z,)),
        ],
        compiler_params=pltpu.CompilerParams(collective_id=0),
    )(x, router_W, route_idx, expert_W, shared_W)


# device time: 34431 ns/iter; 2.7579x vs baseline; 1.3094x over previous
import jax
import jax.numpy as jnp
from jax import lax
from jax.experimental import pallas as pl
from jax.experimental.pallas import tpu as pltpu

N_DEV = 16
PLANE = 4
N_Z = 4
N_TOK = 1024
D_IN = 512
D_OUT = 1024
E_LOCAL = 4
ROWS = N_TOK // N_DEV


def kernel(x, router_W, route_idx, expert_W, shared_W):
    def body(x_ref, rw_ref, ri_ref, ew_ref, sw_ref, out_ref,
             partial_ref, partial_bf_ref, rp1_ref, rp2_ref, scale_ref,
             p1_send_sems, p1_recv_sems, p2_send_sems, p2_recv_sems):
        d = lax.axis_index("i")
        my_z = lax.div(d, PLANE)
        my_w = lax.rem(d, PLANE)

        scores = x_ref[:, :] @ rw_ref[:, :]
        m = jnp.max(scores, axis=-1, keepdims=True)
        p = jnp.exp(scores - m)
        probs = p / jnp.sum(p, axis=-1, keepdims=True)
        route = ri_ref[:, :]
        eids = lax.broadcasted_iota(route.dtype, scores.shape, 1)
        coef = jnp.sum(jnp.where(eids == route, probs, 0.0),
                       axis=-1, keepdims=True)
        scale_ref[:, :] = jnp.concatenate(
            [jnp.where(route == d * E_LOCAL + e, coef, 0.0)
             for e in range(E_LOCAL)], axis=-1)

        barrier = pltpu.get_barrier_semaphore()
        for o in range(1, PLANE):
            wp = lax.rem(my_w + o, PLANE)
            pl.semaphore_signal(barrier, inc=1, device_id=(my_z * PLANE + wp,),
                                device_id_type=pl.DeviceIdType.MESH)
        for o in range(1, N_Z):
            zq = lax.rem(my_z + o, N_Z)
            pl.semaphore_signal(barrier, inc=1, device_id=(zq * PLANE + my_w,),
                                device_id_type=pl.DeviceIdType.MESH)
        pl.semaphore_wait(barrier, 6)

        sends = []

        BLK = PLANE * ROWS
        for k in range(1, N_Z + 1):
            zk = lax.rem(my_z + k, N_Z)
            rs = zk * BLK
            acc = (x_ref[pl.ds(rs, BLK), :] * scale_ref[pl.ds(rs, BLK), 0:1]) @ ew_ref[0]
            for e in range(1, E_LOCAL):
                acc += (x_ref[pl.ds(rs, BLK), :] * scale_ref[pl.ds(rs, BLK), e:e + 1]) @ ew_ref[e]
            partial_ref[pl.ds(rs, BLK), :] = acc
            partial_bf_ref[pl.ds(rs, BLK), :] = acc.astype(jnp.bfloat16)
            for o in range(1, PLANE):
                wp = lax.rem(my_w + o, PLANE)
                dest = my_z * PLANE + wp
                chunk = 4 * zk + wp
                rdma = pltpu.make_async_remote_copy(
                    src_ref=partial_bf_ref.at[pl.ds(chunk * ROWS, ROWS), :],
                    dst_ref=rp1_ref.at[my_w, zk],
                    send_sem=p1_send_sems.at[wp, zk],
                    recv_sem=p1_recv_sems.at[my_w, zk],
                    device_id=(dest,),
                    device_id_type=pl.DeviceIdType.MESH,
                )
                rdma.start()
                sends.append(rdma)

        def accum_zgroup(zk):
            chunk = 4 * zk + my_w
            acc2 = partial_ref[pl.ds(chunk * ROWS, ROWS), :]
            for oo in range(1, PLANE):
                wq = lax.rem(my_w + oo, PLANE)
                recv = pltpu.make_async_remote_copy(
                    src_ref=partial_bf_ref.at[pl.ds(0, ROWS), :],
                    dst_ref=rp1_ref.at[wq, zk],
                    send_sem=p1_send_sems.at[wq, zk],
                    recv_sem=p1_recv_sems.at[wq, zk],
                    device_id=(d,),
                    device_id_type=pl.DeviceIdType.MESH,
                )
                recv.wait_recv()
                acc2 = acc2 + rp1_ref[wq, zk].astype(jnp.float32)
            return acc2

        for o in range(1, N_Z):
            zq = lax.rem(my_z + o, N_Z)
            dest = zq * PLANE + my_w
            acc2 = accum_zgroup(zq)
            partial_bf_ref[pl.ds(dest * ROWS, ROWS), :] = acc2.astype(jnp.bfloat16)
            rdma = pltpu.make_async_remote_copy(
                src_ref=partial_bf_ref.at[pl.ds(dest * ROWS, ROWS), :],
                dst_ref=rp2_ref.at[my_z],
                send_sem=p2_send_sems.at[zq],
                recv_sem=p2_recv_sems.at[my_z],
                device_id=(dest,),
                device_id_type=pl.DeviceIdType.MESH,
            )
            rdma.start()
            sends.append(rdma)

        result = accum_zgroup(my_z)
        result += x_ref[pl.ds(d * ROWS, ROWS), :] @ sw_ref[:, :]

        for o in range(1, N_Z):
            zr = lax.rem(my_z + o, N_Z)
            recv = pltpu.make_async_remote_copy(
                src_ref=partial_bf_ref.at[pl.ds(0, ROWS), :],
                dst_ref=rp2_ref.at[zr],
                send_sem=p2_send_sems.at[zr],
                recv_sem=p2_recv_sems.at[zr],
                device_id=(d,),
                device_id_type=pl.DeviceIdType.MESH,
            )
            recv.wait_recv()
            result += rp2_ref[zr].astype(jnp.float32)
        out_ref[:, :] = result

        for rdma in sends:
            rdma.wait_send()

    return pl.pallas_call(
        body,
        out_shape=jax.ShapeDtypeStruct((ROWS, D_OUT), jnp.float32),
        in_specs=[pl.BlockSpec(memory_space=pltpu.VMEM)] * 5,
        out_specs=pl.BlockSpec(memory_space=pltpu.VMEM),
        scratch_shapes=[
            pltpu.VMEM((N_TOK, D_OUT), jnp.float32),
            pltpu.VMEM((N_TOK, D_OUT), jnp.bfloat16),
            pltpu.VMEM((PLANE, N_Z, ROWS, D_OUT), jnp.bfloat16),
            pltpu.VMEM((N_Z, ROWS, D_OUT), jnp.bfloat16),
            pltpu.VMEM((N_TOK, E_LOCAL), jnp.float32),
            pltpu.SemaphoreType.DMA((PLANE, N_Z)),
            pltpu.SemaphoreType.DMA((PLANE, N_Z)),
            pltpu.SemaphoreType.DMA((N_Z,)),
            pltpu.SemaphoreType.DMA((N_Z,)),
        ],
        compiler_params=pltpu.CompilerParams(collective_id=0),
    )(x, router_W, route_idx, expert_W, shared_W)
